# baseline (speedup 1.0000x reference)
import functools

import numpy as np
import jax
import jax.numpy as jnp
from jax import lax
from jax.experimental import pallas as pl
from jax.experimental.pallas import tpu as pltpu
from jax.experimental.pallas import tpu_sc as plsc


def _pos_table(t, dim):
    omega = np.arange(dim // 2, dtype=np.float64)
    omega = 1.0 / (10000.0 ** (omega / (dim / 2.0)))
    out = np.einsum("p,d->pd", np.arange(t, dtype=np.float64), omega)
    return np.concatenate([np.sin(out), np.cos(out)], axis=-1).astype(np.float32)


def _month_table(d_hid):
    angles = np.arange(0, 13) / (12.0 / (2.0 * np.pi))
    sin_t = np.sin(np.stack([angles] * (d_hid // 2), axis=-1))
    cos_t = np.cos(np.stack([angles] * (d_hid // 2), axis=-1))
    return np.concatenate([sin_t[:-1], cos_t[:-1]], axis=-1).astype(np.float32)


@functools.lru_cache(maxsize=None)
def _make_sc_gather(n_rows, d):
    info = plsc.get_sparse_core_info()
    nc, ns = info.num_cores, info.num_subcores
    nw = nc * ns
    per_w = n_rows // nw
    assert n_rows % nw == 0 and per_w % 8 == 0
    mesh = plsc.VectorSubcoreMesh(core_axis_name="c", subcore_axis_name="s")

    @functools.partial(
        pl.kernel,
        mesh=mesh,
        out_type=jax.ShapeDtypeStruct((n_rows, d), jnp.float32),
        scratch_types=[
            pltpu.VMEM((per_w,), jnp.int32),
            pltpu.VMEM((per_w, d), jnp.float32),
            pltpu.SemaphoreType.DMA,
        ],
    )
    def gather(table_hbm, idx_hbm, out_hbm, idx_v, rows_v, sem):
        wid = lax.axis_index("s") * nc + lax.axis_index("c")
        base = wid * per_w
        pltpu.sync_copy(idx_hbm.at[pl.ds(base, per_w)], idx_v)
        pltpu.async_copy(table_hbm.at[idx_v], rows_v, sem).wait()
        pltpu.sync_copy(rows_v, out_hbm.at[pl.ds(base, per_w)])

    return gather


def _make_tc_body(t, s, n):
    ts = t * s

    def body(tok_ref, mon_ref, pos_ref, ch_ref, out_ref, enc_ref, pt_ref, sem):
        i = pl.program_id(0)
        q3 = pltpu.make_async_copy(
            tok_ref.at[:, :, pl.ds(3 * n, n)], out_ref.at[:, :, pl.ds(3 * n, n)], sem)
        q3.start()

        @pl.when(i == 0)
        def _build_static():
            row_t = lax.broadcasted_iota(jnp.int32, (ts, t), 0) // s
            col_t = lax.broadcasted_iota(jnp.int32, (ts, t), 1)
            pt_ref[...] = (row_t == col_t).astype(jnp.float32)
            row_s = lax.broadcasted_iota(jnp.int32, (ts, s), 0) % s
            col_s = lax.broadcasted_iota(jnp.int32, (ts, s), 1)
            ps = (row_s == col_s).astype(jnp.float32)
            enc_ref[:, 0:n] = jnp.dot(
                ps, ch_ref[...], preferred_element_type=jnp.float32)
            enc_ref[:, n:2 * n] = jnp.dot(
                pt_ref[...], pos_ref[...], preferred_element_type=jnp.float32)

        enc = enc_ref[...]
        out_ref[:, :, 0:2 * n] = tok_ref[:, :, 0:2 * n] + enc[None, :, :]
        bb = tok_ref.shape[0]
        for k in range(bb):
            mon_k = jnp.dot(pt_ref[...], mon_ref[k],
                            preferred_element_type=jnp.float32)
            out_ref[k, :, 2 * n:3 * n] = tok_ref[k, :, 2 * n:3 * n] + mon_k
        q3.wait()

    return body


def kernel(tokens, timestamps, ch_embed, patch_size):
    b, t, s, d = tokens.shape
    n = d // 4
    ts = t * s
    pos = jnp.asarray(_pos_table(t, n))
    mtab = jnp.asarray(_month_table(n))
    months = timestamps[:, :, 1].reshape(-1)
    month_e = _make_sc_gather(b * t, n)(mtab, months).reshape(b, t, n)
    tok_rows = tokens.reshape(b, ts, d)
    bb = 8
    out = pl.pallas_call(
        _make_tc_body(t, s, n),
        grid=(b // bb,),
        in_specs=[
            pl.BlockSpec((bb, ts, d), lambda i: (i, 0, 0)),
            pl.BlockSpec((bb, t, n), lambda i: (i, 0, 0)),
            pl.BlockSpec((t, n), lambda i: (0, 0)),
            pl.BlockSpec((s, n), lambda i: (0, 0)),
        ],
        out_specs=pl.BlockSpec((bb, ts, d), lambda i: (i, 0, 0)),
        out_shape=jax.ShapeDtypeStruct((b, ts, d), jnp.float32),
        scratch_shapes=[
            pltpu.VMEM((ts, 2 * n), jnp.float32),
            pltpu.VMEM((ts, t), jnp.float32),
            pltpu.SemaphoreType.DMA,
        ],
    )(tok_rows, month_e, pos, ch_embed)
    return out.reshape(b, t, s, d)


# speedup vs baseline: 1.0034x; 1.0034x over previous
import functools

import numpy as np
import jax
import jax.numpy as jnp
from jax import lax
from jax.experimental import pallas as pl
from jax.experimental.pallas import tpu as pltpu
from jax.experimental.pallas import tpu_sc as plsc


def _pos_table(t, dim):
    omega = np.arange(dim // 2, dtype=np.float64)
    omega = 1.0 / (10000.0 ** (omega / (dim / 2.0)))
    out = np.einsum("p,d->pd", np.arange(t, dtype=np.float64), omega)
    return np.concatenate([np.sin(out), np.cos(out)], axis=-1).astype(np.float32)


def _month_table(d_hid):
    angles = np.arange(0, 13) / (12.0 / (2.0 * np.pi))
    sin_t = np.sin(np.stack([angles] * (d_hid // 2), axis=-1))
    cos_t = np.cos(np.stack([angles] * (d_hid // 2), axis=-1))
    return np.concatenate([sin_t[:-1], cos_t[:-1]], axis=-1).astype(np.float32)


@functools.lru_cache(maxsize=None)
def _make_sc_gather(n_rows, d):
    info = plsc.get_sparse_core_info()
    nc, ns = info.num_cores, info.num_subcores
    nw = nc * ns
    per_w = n_rows // nw
    assert n_rows % nw == 0 and per_w % 8 == 0
    mesh = plsc.VectorSubcoreMesh(core_axis_name="c", subcore_axis_name="s")

    @functools.partial(
        pl.kernel,
        mesh=mesh,
        out_type=jax.ShapeDtypeStruct((n_rows, d), jnp.float32),
        scratch_types=[
            pltpu.VMEM((per_w,), jnp.int32),
            pltpu.VMEM((per_w, d), jnp.float32),
            pltpu.SemaphoreType.DMA,
        ],
    )
    def gather(table_hbm, idx_hbm, out_hbm, idx_v, rows_v, sem):
        wid = lax.axis_index("s") * nc + lax.axis_index("c")
        base = wid * per_w
        pltpu.sync_copy(idx_hbm.at[pl.ds(base, per_w)], idx_v)
        pltpu.async_copy(table_hbm.at[idx_v], rows_v, sem).wait()
        pltpu.sync_copy(rows_v, out_hbm.at[pl.ds(base, per_w)])

    return gather


def _tc_body(tok_ref, mon_ref, pos_ref, ch_ref, out_ref):
    n = ch_ref.shape[-1]
    tok = tok_ref[...]
    out_ref[..., 0:n] = tok[..., 0:n] + ch_ref[...][None, None, :, :]
    out_ref[..., n:2 * n] = tok[..., n:2 * n] + pos_ref[...][None, :, None, :]
    out_ref[..., 2 * n:3 * n] = tok[..., 2 * n:3 * n] + mon_ref[...][:, :, None, :]
    out_ref[..., 3 * n:] = tok[..., 3 * n:]


def kernel(tokens, timestamps, ch_embed, patch_size):
    b, t, s, d = tokens.shape
    n = d // 4
    pos = jnp.asarray(_pos_table(t, n))
    mtab = jnp.asarray(_month_table(n))
    months = timestamps[:, :, 1].reshape(-1)
    month_e = _make_sc_gather(b * t, n)(mtab, months).reshape(b, t, n)
    bb = 8
    return pl.pallas_call(
        _tc_body,
        grid=(b // bb,),
        in_specs=[
            pl.BlockSpec((bb, t, s, d), lambda i: (i, 0, 0, 0)),
            pl.BlockSpec((bb, t, n), lambda i: (i, 0, 0)),
            pl.BlockSpec((t, n), lambda i: (0, 0)),
            pl.BlockSpec((s, n), lambda i: (0, 0)),
        ],
        out_specs=pl.BlockSpec((bb, t, s, d), lambda i: (i, 0, 0, 0)),
        out_shape=jax.ShapeDtypeStruct((b, t, s, d), jnp.float32),
    )(tokens, month_e, pos, ch_embed)
